# baseline (speedup 1.0000x reference)
import functools

import jax
import jax.numpy as jnp
import numpy as np
from jax import lax
from jax.experimental import pallas as pl
from jax.experimental.pallas import tpu as pltpu
from jax.experimental.pallas import tpu_sc as plsc

_CARDS = [100000] * 26
_F = len(_CARDS)
_D = 32
_B = 16384
_R = sum(_CARDS)

_NC, _NS, _L = 2, 16, 16
_NW = _NC * _NS

_TCOLS = -(-_R // 128)
_RP = _TCOLS * 32
_TC_PER_W = -(-_TCOLS // _NW)

_PAIRS = _F * (_B // 128)
_PER_W = _PAIRS // _NW


def _pack_body(emb_hbm, out_hbm, in_v, out_v, sem_i, sem_o):
    cid = lax.axis_index("c")
    sid = lax.axis_index("s")
    wid = sid * _NC + cid
    tc0 = wid * _TC_PER_W
    cnt = jnp.minimum(_TC_PER_W, _TCOLS - tc0)
    lanes = lax.iota(jnp.int32, _L)

    def fire_in(t):
        return pltpu.async_copy(
            emb_hbm.at[:, pl.ds((tc0 + t) * 128, 128)], in_v.at[t & 1], sem_i)

    def fire_out(t):
        return pltpu.async_copy(
            out_v.at[t & 1], out_hbm.at[pl.ds((tc0 + t) * 32, 32), :], sem_o)

    fire_in(0)
    fire_in(1)

    def step(t, carry):
        b = t & 1
        pltpu.make_async_copy(
            emb_hbm.at[:, pl.ds((tc0 + t) * 128, 128)], in_v.at[b],
            sem_i).wait()
        def line(q, c2):
            for k in range(8):
                val = plsc.load_gather(
                    in_v, [jnp.full((_L,), b, jnp.int32),
                           lanes + 16 * (k & 1),
                           jnp.full((_L,), 4 * q + k // 2, jnp.int32)])
                out_v[b, q, pl.ds(k * _L, _L)] = val
            return c2
        lax.fori_loop(0, 32, line, 0)

        @pl.when(t >= 2)
        def _():
            pltpu.make_async_copy(
                out_v.at[b], out_hbm.at[pl.ds((tc0 + t - 2) * 32, 32), :],
                sem_o).wait()
        fire_out(t)

        @pl.when(t + 2 < cnt)
        def _():
            fire_in(t + 2)
        return carry

    lax.fori_loop(0, cnt, step, 0)
    pltpu.make_async_copy(
        out_v.at[(cnt - 2) & 1],
        out_hbm.at[pl.ds((tc0 + cnt - 2) * 32, 32), :], sem_o).wait()
    pltpu.make_async_copy(
        out_v.at[(cnt - 1) & 1],
        out_hbm.at[pl.ds((tc0 + cnt - 1) * 32, 32), :], sem_o).wait()


def _gather_body(x_hbm, bias_hbm, emb_hbm, out_hbm,
                 x_v, idx_v, slot_v, wide_v, tiles_v, bias_v, sem_g, sem_f):
    cid = lax.axis_index("c")
    sid = lax.axis_index("s")
    wid = sid * _NC + cid
    out2 = out_hbm.reshape(_F * _D, _B)

    pltpu.sync_copy(x_hbm.at[pl.ds(wid * _PER_W * 128, _PER_W * 128)], x_v)
    pltpu.sync_copy(bias_hbm, bias_v)
    lanes = lax.iota(jnp.int32, _L)

    def prep(t):
        p = wid * _PER_W + t
        off = ((p // 128) * 100000).astype(jnp.float32)
        b = t & 1
        for k in range(8):
            r = (x_v[pl.ds(t * 128 + k * _L, _L)] + off).astype(jnp.int32)
            idx_v[b, pl.ds(k * _L, _L)] = r >> 2
            slot_v[pl.ds(b * 128 + k * _L, _L)] = (r & 3) << 5
        pltpu.async_copy(emb_hbm.at[idx_v.at[b]], wide_v.at[b], sem_g)

    prep(0)

    def step(t, carry):
        b = t & 1
        p = wid * _PER_W + t
        f = p // 128
        tc = p % 128

        @pl.when(t + 1 < _PER_W)
        def _():
            prep(t + 1)
        pltpu.make_async_copy(emb_hbm.at[idx_v.at[b]], wide_v.at[b],
                              sem_g).wait()

        @pl.when(t >= 2)
        def _():
            for tr in range(4):
                pltpu.make_async_copy(
                    tiles_v.at[b, pl.ds(tr * 8, 8), :],
                    out2.at[pl.ds(((p - 2) // 128) * _D + tr * 8, 8),
                            pl.ds(((p - 2) % 128) * 128, 128)],
                    sem_f).wait()

        def pick(j, c2):
            bvec = bias_v[pl.ds((f * _D + j) * _L, _L)]
            for k in range(8):
                val = plsc.load_gather(
                    wide_v, [jnp.full((_L,), b, jnp.int32),
                             k * _L + lanes,
                             slot_v[pl.ds(b * 128 + k * _L, _L)] + j])
                tiles_v[b, j, pl.ds(k * _L, _L)] = val + bvec
            return c2
        lax.fori_loop(0, _D, pick, 0)

        for tr in range(4):
            pltpu.async_copy(
                tiles_v.at[b, pl.ds(tr * 8, 8), :],
                out2.at[pl.ds(f * _D + tr * 8, 8), pl.ds(tc * 128, 128)],
                sem_f)
        return carry

    lax.fori_loop(0, _PER_W, step, 0)
    for t in (_PER_W - 2, _PER_W - 1):
        p = wid * _PER_W + t
        for tr in range(4):
            pltpu.make_async_copy(
                tiles_v.at[t & 1, pl.ds(tr * 8, 8), :],
                out2.at[pl.ds((p // 128) * _D + tr * 8, 8),
                        pl.ds((p % 128) * 128, 128)],
                sem_f).wait()


@jax.jit
def kernel(x, emb, bias):
    x_flat = x.T.reshape(_F * _B)
    bias_bc = jnp.repeat(bias.reshape(_F * _D), _L)

    mesh = plsc.VectorSubcoreMesh(core_axis_name="c", subcore_axis_name="s")
    params = pltpu.CompilerParams(use_tc_tiling_on_sc=True,
                                  needs_layout_passes=False)
    pack = functools.partial(
        pl.kernel,
        out_type=jax.ShapeDtypeStruct((_RP, 128), jnp.float32),
        mesh=mesh,
        compiler_params=params,
        scratch_types=[
            pltpu.VMEM((2, _D, 128), jnp.float32),
            pltpu.VMEM((2, _D, 128), jnp.float32),
            pltpu.SemaphoreType.DMA,
            pltpu.SemaphoreType.DMA,
        ],
    )(_pack_body)
    gather = functools.partial(
        pl.kernel,
        out_type=jax.ShapeDtypeStruct((_F, _D, _B), jnp.float32),
        mesh=mesh,
        compiler_params=params,
        scratch_types=[
            pltpu.VMEM((_PER_W * 128,), jnp.float32),
            pltpu.VMEM((2, 128), jnp.int32),
            pltpu.VMEM((2 * 128,), jnp.int32),
            pltpu.VMEM((2, 128, 128), jnp.float32),
            pltpu.VMEM((2, _D, 128), jnp.float32),
            pltpu.VMEM((_F * _D * _L,), jnp.float32),
            pltpu.SemaphoreType.DMA,
            pltpu.SemaphoreType.DMA,
        ],
    )(_gather_body)

    t_pack = pack(emb.T)
    out_t = gather(x_flat, bias_bc, t_pack)
    return jnp.transpose(out_t, (2, 0, 1))


# speedup vs baseline: 1.0617x; 1.0617x over previous
import functools

import jax
import jax.numpy as jnp
import numpy as np
from jax import lax
from jax.experimental import pallas as pl
from jax.experimental.pallas import tpu as pltpu
from jax.experimental.pallas import tpu_sc as plsc

_CARDS = [100000] * 26
_F = len(_CARDS)
_D = 32
_B = 16384
_R = sum(_CARDS)

_NC, _NS, _L = 2, 16, 16
_NW = _NC * _NS

_TCOLS = -(-_R // 128)
_RP = _TCOLS * 32
_TC_PER_W = -(-_TCOLS // _NW)

_PAIRS = _F * (_B // 128)
_PER_W = _PAIRS // _NW


def _pack_body(emb_hbm, out_hbm, in_v, out_v, sem_i, sem_o):
    cid = lax.axis_index("c")
    sid = lax.axis_index("s")
    wid = sid * _NC + cid
    tc0 = wid * _TC_PER_W
    cnt = jnp.minimum(_TC_PER_W, _TCOLS - tc0)
    lanes = lax.iota(jnp.int32, _L)

    def fire_in(t):
        return pltpu.async_copy(
            emb_hbm.at[:, pl.ds((tc0 + t) * 128, 128)], in_v.at[t & 1], sem_i)

    def fire_out(t):
        return pltpu.async_copy(
            out_v.at[t & 1], out_hbm.at[pl.ds((tc0 + t) * 32, 32), :], sem_o)

    fire_in(0)
    fire_in(1)

    def step(t, carry):
        b = t & 1
        pltpu.make_async_copy(
            emb_hbm.at[:, pl.ds((tc0 + t) * 128, 128)], in_v.at[b],
            sem_i).wait()
        bvec = jnp.full((_L,), b, jnp.int32)
        for q in range(32):
            for k in range(8):
                val = plsc.load_gather(
                    in_v, [bvec, lanes + 16 * (k & 1),
                           jnp.full((_L,), 4 * q + k // 2, jnp.int32)])
                out_v[b, q, pl.ds(k * _L, _L)] = val

        @pl.when(t >= 2)
        def _():
            pltpu.make_async_copy(
                out_v.at[b], out_hbm.at[pl.ds((tc0 + t - 2) * 32, 32), :],
                sem_o).wait()
        fire_out(t)

        @pl.when(t + 2 < cnt)
        def _():
            fire_in(t + 2)
        return carry

    lax.fori_loop(0, cnt, step, 0)
    pltpu.make_async_copy(
        out_v.at[(cnt - 2) & 1],
        out_hbm.at[pl.ds((tc0 + cnt - 2) * 32, 32), :], sem_o).wait()
    pltpu.make_async_copy(
        out_v.at[(cnt - 1) & 1],
        out_hbm.at[pl.ds((tc0 + cnt - 1) * 32, 32), :], sem_o).wait()


def _gather_body(x_hbm, bias_hbm, emb_hbm, out_hbm,
                 x_v, idx_v, slot_v, wide_v, tiles_v, bias_v, sem_g, sem_f):
    cid = lax.axis_index("c")
    sid = lax.axis_index("s")
    wid = sid * _NC + cid
    out2 = out_hbm.reshape(_F * _D, _B)

    pltpu.sync_copy(x_hbm.at[pl.ds(wid * _PER_W * 128, _PER_W * 128)], x_v)
    pltpu.sync_copy(bias_hbm, bias_v)
    lanes = lax.iota(jnp.int32, _L)

    def prep(t):
        p = wid * _PER_W + t
        off = ((p // 128) * 100000).astype(jnp.float32)
        b = t & 1
        for k in range(8):
            r = (x_v[pl.ds(t * 128 + k * _L, _L)] + off).astype(jnp.int32)
            idx_v[b, pl.ds(k * _L, _L)] = r >> 2
            slot_v[pl.ds(b * 128 + k * _L, _L)] = (r & 3) << 5
        pltpu.async_copy(emb_hbm.at[idx_v.at[b]], wide_v.at[b], sem_g)

    prep(0)

    def step(t, carry):
        b = t & 1
        p = wid * _PER_W + t
        f = p // 128
        tc = p % 128

        @pl.when(t + 1 < _PER_W)
        def _():
            prep(t + 1)
        pltpu.make_async_copy(emb_hbm.at[idx_v.at[b]], wide_v.at[b],
                              sem_g).wait()

        @pl.when(t >= 2)
        def _():
            for tr in range(4):
                pltpu.make_async_copy(
                    tiles_v.at[b, pl.ds(tr * 8, 8), :],
                    out2.at[pl.ds(((p - 2) // 128) * _D + tr * 8, 8),
                            pl.ds(((p - 2) % 128) * 128, 128)],
                    sem_f).wait()

        buf = jnp.full((_L,), b, jnp.int32)
        slots = [slot_v[pl.ds(b * 128 + k * _L, _L)] for k in range(8)]
        for j in range(_D):
            bvec = bias_v[pl.ds((f * _D + j) * _L, _L)]
            for k in range(8):
                val = plsc.load_gather(
                    wide_v, [buf, k * _L + lanes, slots[k] + j])
                tiles_v[b, j, pl.ds(k * _L, _L)] = val + bvec

        for tr in range(4):
            pltpu.async_copy(
                tiles_v.at[b, pl.ds(tr * 8, 8), :],
                out2.at[pl.ds(f * _D + tr * 8, 8), pl.ds(tc * 128, 128)],
                sem_f)
        return carry

    lax.fori_loop(0, _PER_W, step, 0)
    for t in (_PER_W - 2, _PER_W - 1):
        p = wid * _PER_W + t
        for tr in range(4):
            pltpu.make_async_copy(
                tiles_v.at[t & 1, pl.ds(tr * 8, 8), :],
                out2.at[pl.ds((p // 128) * _D + tr * 8, 8),
                        pl.ds((p % 128) * 128, 128)],
                sem_f).wait()


@jax.jit
def kernel(x, emb, bias):
    x_flat = x.T.reshape(_F * _B)
    bias_bc = jnp.repeat(bias.reshape(_F * _D), _L)

    mesh = plsc.VectorSubcoreMesh(core_axis_name="c", subcore_axis_name="s")
    params = pltpu.CompilerParams(use_tc_tiling_on_sc=True,
                                  needs_layout_passes=False)
    pack = functools.partial(
        pl.kernel,
        out_type=jax.ShapeDtypeStruct((_RP, 128), jnp.float32),
        mesh=mesh,
        compiler_params=params,
        scratch_types=[
            pltpu.VMEM((2, _D, 128), jnp.float32),
            pltpu.VMEM((2, _D, 128), jnp.float32),
            pltpu.SemaphoreType.DMA,
            pltpu.SemaphoreType.DMA,
        ],
    )(_pack_body)
    gather = functools.partial(
        pl.kernel,
        out_type=jax.ShapeDtypeStruct((_F, _D, _B), jnp.float32),
        mesh=mesh,
        compiler_params=params,
        scratch_types=[
            pltpu.VMEM((_PER_W * 128,), jnp.float32),
            pltpu.VMEM((2, 128), jnp.int32),
            pltpu.VMEM((2 * 128,), jnp.int32),
            pltpu.VMEM((2, 128, 128), jnp.float32),
            pltpu.VMEM((2, _D, 128), jnp.float32),
            pltpu.VMEM((_F * _D * _L,), jnp.float32),
            pltpu.SemaphoreType.DMA,
            pltpu.SemaphoreType.DMA,
        ],
    )(_gather_body)

    t_pack = pack(emb.T)
    out_t = gather(x_flat, bias_bc, t_pack)
    return jnp.transpose(out_t, (2, 0, 1))


# speedup vs baseline: 2.0086x; 1.8920x over previous
import functools

import jax
import jax.numpy as jnp
import numpy as np
from jax import lax
from jax.experimental import pallas as pl
from jax.experimental.pallas import tpu as pltpu
from jax.experimental.pallas import tpu_sc as plsc

_CARDS = [100000] * 26
_F = len(_CARDS)
_D = 32
_B = 16384
_R = sum(_CARDS)

_NC, _NS, _L = 2, 16, 16
_NW = _NC * _NS

_TCOLS = -(-_R // 128)
_RP = _TCOLS * 32
_TC_PER_W = -(-_TCOLS // _NW)

_PAIRS = _F * (_B // 128)
_PER_W = _PAIRS // _NW


def _pack_body(emb_hbm, out_hbm, in_v, out_v, sem_i, sem_o):
    cid = lax.axis_index("c")
    sid = lax.axis_index("s")
    wid = sid * _NC + cid
    tc0 = wid * _TC_PER_W
    cnt = jnp.minimum(_TC_PER_W, _TCOLS - tc0)
    lanes = lax.iota(jnp.int32, _L)

    def fire_in(t):
        return pltpu.async_copy(
            emb_hbm.at[:, pl.ds((tc0 + t) * 128, 128)], in_v.at[t & 1], sem_i)

    def fire_out(t):
        return pltpu.async_copy(
            out_v.at[t & 1], out_hbm.at[pl.ds((tc0 + t) * 32, 32), :], sem_o)

    fire_in(0)
    fire_in(1)

    def step(t, carry):
        b = t & 1
        pltpu.make_async_copy(
            emb_hbm.at[:, pl.ds((tc0 + t) * 128, 128)], in_v.at[b],
            sem_i).wait()
        bvec = jnp.full((_L,), b, jnp.int32)

        @plsc.parallel_loop(0, 32, unroll=4)
        def _line(q):
            for k in range(8):
                val = plsc.load_gather(
                    in_v, [bvec, lanes + 16 * (k & 1),
                           jnp.full((_L,), 4 * q + k // 2, jnp.int32)])
                out_v[b, q, pl.ds(k * _L, _L)] = val

        @pl.when(t >= 2)
        def _():
            pltpu.make_async_copy(
                out_v.at[b], out_hbm.at[pl.ds((tc0 + t - 2) * 32, 32), :],
                sem_o).wait()
        fire_out(t)

        @pl.when(t + 2 < cnt)
        def _():
            fire_in(t + 2)
        return carry

    lax.fori_loop(0, cnt, step, 0)
    pltpu.make_async_copy(
        out_v.at[(cnt - 2) & 1],
        out_hbm.at[pl.ds((tc0 + cnt - 2) * 32, 32), :], sem_o).wait()
    pltpu.make_async_copy(
        out_v.at[(cnt - 1) & 1],
        out_hbm.at[pl.ds((tc0 + cnt - 1) * 32, 32), :], sem_o).wait()


def _gather_body(x_hbm, bias_hbm, emb_hbm, out_hbm,
                 x_v, idx_v, slot_v, wide_v, tiles_v, bias_v, sem_g, sem_f):
    cid = lax.axis_index("c")
    sid = lax.axis_index("s")
    wid = sid * _NC + cid
    out2 = out_hbm.reshape(_F * _D, _B)

    pltpu.sync_copy(x_hbm.at[pl.ds(wid * _PER_W * 128, _PER_W * 128)], x_v)
    pltpu.sync_copy(bias_hbm, bias_v)
    lanes = lax.iota(jnp.int32, _L)

    def prep(t):
        p = wid * _PER_W + t
        off = ((p // 128) * 100000).astype(jnp.float32)
        b = t & 1
        for k in range(8):
            r = (x_v[pl.ds(t * 128 + k * _L, _L)] + off).astype(jnp.int32)
            idx_v[b, pl.ds(k * _L, _L)] = r >> 2
            slot_v[pl.ds(b * 128 + k * _L, _L)] = (r & 3) << 5
        pltpu.async_copy(emb_hbm.at[idx_v.at[b]], wide_v.at[b], sem_g)

    prep(0)

    def step(t, carry):
        b = t & 1
        p = wid * _PER_W + t
        f = p // 128
        tc = p % 128

        @pl.when(t + 1 < _PER_W)
        def _():
            prep(t + 1)
        pltpu.make_async_copy(emb_hbm.at[idx_v.at[b]], wide_v.at[b],
                              sem_g).wait()

        @pl.when(t >= 2)
        def _():
            for tr in range(4):
                pltpu.make_async_copy(
                    tiles_v.at[b, pl.ds(tr * 8, 8), :],
                    out2.at[pl.ds(((p - 2) // 128) * _D + tr * 8, 8),
                            pl.ds(((p - 2) % 128) * 128, 128)],
                    sem_f).wait()

        buf = jnp.full((_L,), b, jnp.int32)
        slots = [slot_v[pl.ds(b * 128 + k * _L, _L)] for k in range(8)]

        @plsc.parallel_loop(0, _D, unroll=4)
        def _pick(j):
            bvec = bias_v[pl.ds((f * _D + j) * _L, _L)]
            for k in range(8):
                val = plsc.load_gather(
                    wide_v, [buf, k * _L + lanes, slots[k] + j])
                tiles_v[b, j, pl.ds(k * _L, _L)] = val + bvec

        for tr in range(4):
            pltpu.async_copy(
                tiles_v.at[b, pl.ds(tr * 8, 8), :],
                out2.at[pl.ds(f * _D + tr * 8, 8), pl.ds(tc * 128, 128)],
                sem_f)
        return carry

    lax.fori_loop(0, _PER_W, step, 0)
    for t in (_PER_W - 2, _PER_W - 1):
        p = wid * _PER_W + t
        for tr in range(4):
            pltpu.make_async_copy(
                tiles_v.at[t & 1, pl.ds(tr * 8, 8), :],
                out2.at[pl.ds((p // 128) * _D + tr * 8, 8),
                        pl.ds((p % 128) * 128, 128)],
                sem_f).wait()


@jax.jit
def kernel(x, emb, bias):
    x_flat = x.T.reshape(_F * _B)
    bias_bc = jnp.repeat(bias.reshape(_F * _D), _L)

    mesh = plsc.VectorSubcoreMesh(core_axis_name="c", subcore_axis_name="s")
    params = pltpu.CompilerParams(use_tc_tiling_on_sc=True,
                                  needs_layout_passes=False)
    pack = functools.partial(
        pl.kernel,
        out_type=jax.ShapeDtypeStruct((_RP, 128), jnp.float32),
        mesh=mesh,
        compiler_params=params,
        scratch_types=[
            pltpu.VMEM((2, _D, 128), jnp.float32),
            pltpu.VMEM((2, _D, 128), jnp.float32),
            pltpu.SemaphoreType.DMA,
            pltpu.SemaphoreType.DMA,
        ],
    )(_pack_body)
    gather = functools.partial(
        pl.kernel,
        out_type=jax.ShapeDtypeStruct((_F, _D, _B), jnp.float32),
        mesh=mesh,
        compiler_params=params,
        scratch_types=[
            pltpu.VMEM((_PER_W * 128,), jnp.float32),
            pltpu.VMEM((2, 128), jnp.int32),
            pltpu.VMEM((2 * 128,), jnp.int32),
            pltpu.VMEM((2, 128, 128), jnp.float32),
            pltpu.VMEM((2, _D, 128), jnp.float32),
            pltpu.VMEM((_F * _D * _L,), jnp.float32),
            pltpu.SemaphoreType.DMA,
            pltpu.SemaphoreType.DMA,
        ],
    )(_gather_body)

    t_pack = pack(emb.T)
    out_t = gather(x_flat, bias_bc, t_pack)
    return jnp.transpose(out_t, (2, 0, 1))
